# Initial kernel scaffold; baseline (speedup 1.0000x reference)
import jax
import jax.numpy as jnp
from jax.experimental import pallas as pl


def kernel(emo_vec, image_tokens, W_emo, b_emo, W_img, b_img, style_dict):
    raise NotImplementedError("write your pallas kernel here")


# speedup vs baseline: 1.8551x; 1.8551x over previous
import functools

import jax
import jax.numpy as jnp
from jax import lax
from jax.experimental import pallas as pl
from jax.experimental.pallas import tpu as pltpu
from jax.experimental.pallas import tpu_sc as plsc

_B = 64
_IN_DIM = 8
_IMG_DIM = 1152
_TOK = 64
_EMB = 2048
_K = 512
_NTOK = 576
_HALF = 1024

_TB = 24
_KB = 8


def _query_body(img_ref, emo_ref, wemo_ref, bemo_ref, wimg_ref, bimg_ref,
                q_ref, acc_ref):
    g = pl.program_id(0)

    @pl.when(g == 0)
    def _init():
        acc_ref[...] = jnp.zeros_like(acc_ref)

    acc_ref[...] += jnp.sum(img_ref[...], axis=1)

    @pl.when(g == pl.num_programs(0) - 1)
    def _epilogue():
        pooled = acc_ref[...] * (1.0 / _NTOK)
        q_img = lax.dot_general(
            pooled, wimg_ref[...], (((1,), (1,)), ((), ())),
            preferred_element_type=jnp.float32,
            precision=lax.Precision.HIGHEST) + bimg_ref[...]
        q_emo = lax.dot_general(
            emo_ref[...], wemo_ref[...], (((1,), (1,)), ((), ())),
            preferred_element_type=jnp.float32,
            precision=lax.Precision.HIGHEST) + bemo_ref[...]
        q = jnp.concatenate([q_emo, q_img], axis=1)
        q_ref[...] = q - jnp.mean(q, axis=1, keepdims=True)


def _scores_body(style_ref, q_ref, out_ref):
    x = style_ref[...]
    cs = jnp.sum(x, axis=1)
    norm = jnp.sqrt(jnp.sum(cs * cs, axis=1, keepdims=True))
    denom = jnp.maximum(norm, _TOK * 1e-6)
    s = lax.dot_general(
        cs, q_ref[...], (((1,), (1,)), ((), ())),
        preferred_element_type=jnp.float32,
        precision=lax.Precision.HIGHEST)
    out_ref[...] = s / denom


def _argmax_body(scores_ref, idx_ref, rows_ref):
    s = scores_ref[...]
    m = jnp.max(s, axis=0, keepdims=True)
    kiota = lax.broadcasted_iota(jnp.int32, (_K, _B), 0)
    cand = jnp.where(s == m, kiota, _K)
    idx = jnp.min(cand, axis=0, keepdims=True)
    idx_ref[...] = idx
    eye = (lax.broadcasted_iota(jnp.int32, (_B, _B), 0) ==
           lax.broadcasted_iota(jnp.int32, (_B, _B), 1)).astype(jnp.float32)
    idx_col = lax.dot_general(
        eye, idx.astype(jnp.float32), (((1,), (1,)), ((), ())),
        preferred_element_type=jnp.float32,
        precision=lax.Precision.HIGHEST)
    tiota = lax.broadcasted_iota(jnp.int32, (_B, _TOK), 1)
    rows_ref[...] = idx_col.astype(jnp.int32) * _TOK + tiota


@functools.lru_cache(maxsize=1)
def _make_sc_gather():
    NC, NS = 2, 16
    NW = NC * NS
    NROWS = _B * _TOK
    CH = 16
    rows_per_w = NROWS // NW
    n_ch = rows_per_w // CH

    @functools.partial(
        pl.kernel,
        out_type=jax.ShapeDtypeStruct((NROWS, _EMB), jnp.float32),
        mesh=plsc.VectorSubcoreMesh(core_axis_name="c", subcore_axis_name="s"),
        scratch_types=[
            pltpu.VMEM((CH,), jnp.int32),
            pltpu.VMEM((CH, _EMB), jnp.float32),
            pltpu.SemaphoreType.DMA,
        ],
    )
    def gather_kernel(table_hbm, rows_hbm, out_hbm, idxg, buf, sem):
        wid = lax.axis_index("s") * NC + lax.axis_index("c")
        for j in range(n_ch):
            base = wid * rows_per_w + j * CH
            pltpu.sync_copy(rows_hbm.at[pl.ds(base, CH)], idxg)
            pltpu.async_copy(table_hbm.at[idxg], buf, sem).wait()
            pltpu.sync_copy(buf, out_hbm.at[pl.ds(base, CH)])

    return gather_kernel


def kernel(emo_vec, image_tokens, W_emo, b_emo, W_img, b_img, style_dict):
    qc = pl.pallas_call(
        _query_body,
        grid=(_NTOK // _TB,),
        in_specs=[
            pl.BlockSpec((_B, _TB, _IMG_DIM), lambda g: (0, g, 0)),
            pl.BlockSpec((_B, _IN_DIM), lambda g: (0, 0)),
            pl.BlockSpec((_HALF, _IN_DIM), lambda g: (0, 0)),
            pl.BlockSpec((1, _HALF), lambda g: (0, 0)),
            pl.BlockSpec((_HALF, _IMG_DIM), lambda g: (0, 0)),
            pl.BlockSpec((1, _HALF), lambda g: (0, 0)),
        ],
        out_specs=pl.BlockSpec((_B, _EMB), lambda g: (0, 0)),
        out_shape=jax.ShapeDtypeStruct((_B, _EMB), jnp.float32),
        scratch_shapes=[pltpu.VMEM((_B, _IMG_DIM), jnp.float32)],
    )(image_tokens, emo_vec, W_emo, b_emo.reshape(1, _HALF), W_img,
      b_img.reshape(1, _HALF))

    scores_t = pl.pallas_call(
        _scores_body,
        grid=(_K // _KB,),
        in_specs=[
            pl.BlockSpec((_KB, _TOK, _EMB), lambda g: (g, 0, 0)),
            pl.BlockSpec((_B, _EMB), lambda g: (0, 0)),
        ],
        out_specs=pl.BlockSpec((_KB, _B), lambda g: (g, 0)),
        out_shape=jax.ShapeDtypeStruct((_K, _B), jnp.float32),
    )(style_dict, qc)

    idx2d, rows2d = pl.pallas_call(
        _argmax_body,
        in_specs=[pl.BlockSpec((_K, _B), lambda: (0, 0))],
        out_specs=[pl.BlockSpec((1, _B), lambda: (0, 0)),
                   pl.BlockSpec((_B, _TOK), lambda: (0, 0))],
        out_shape=[jax.ShapeDtypeStruct((1, _B), jnp.int32),
                   jax.ShapeDtypeStruct((_B, _TOK), jnp.int32)],
    )(scores_t)

    indices = idx2d.reshape(_B)
    table = style_dict.reshape(_K * _TOK, _EMB)
    rows = rows2d.reshape(_B * _TOK)
    style = _make_sc_gather()(table, rows).reshape(_B, _TOK, _EMB)
    return style, indices


# speedup vs baseline: 1.9105x; 1.0298x over previous
import functools

import jax
import jax.numpy as jnp
from jax import lax
from jax.experimental import pallas as pl
from jax.experimental.pallas import tpu as pltpu
from jax.experimental.pallas import tpu_sc as plsc

_B = 64
_IN_DIM = 8
_IMG_DIM = 1152
_TOK = 64
_EMB = 2048
_K = 512
_NTOK = 576
_HALF = 1024

_TB = 24
_KB = 8


def _query_body(img_ref, emo_ref, wemo_ref, bemo_ref, wimg_ref, bimg_ref,
                q_ref, acc_ref):
    g = pl.program_id(0)

    @pl.when(g == 0)
    def _init():
        acc_ref[...] = jnp.zeros_like(acc_ref)

    acc_ref[...] += (img_ref[:, 0:8, :] + img_ref[:, 8:16, :] +
                     img_ref[:, 16:24, :])

    @pl.when(g == pl.num_programs(0) - 1)
    def _epilogue():
        pooled = jnp.sum(acc_ref[...], axis=1) * (1.0 / _NTOK)
        q_img = lax.dot_general(
            pooled, wimg_ref[...], (((1,), (1,)), ((), ())),
            preferred_element_type=jnp.float32,
            precision=lax.Precision.HIGHEST) + bimg_ref[...]
        q_emo = lax.dot_general(
            emo_ref[...], wemo_ref[...], (((1,), (1,)), ((), ())),
            preferred_element_type=jnp.float32,
            precision=lax.Precision.HIGHEST) + bemo_ref[...]
        q = jnp.concatenate([q_emo, q_img], axis=1)
        q_ref[...] = q - jnp.mean(q, axis=1, keepdims=True)


def _scores_body(style_ref, q_ref, out_ref):
    x = style_ref[...]
    cs = jnp.sum(x, axis=1)
    norm = jnp.sqrt(jnp.sum(cs * cs, axis=1, keepdims=True))
    denom = jnp.maximum(norm, _TOK * 1e-6)
    s = lax.dot_general(
        cs, q_ref[...], (((1,), (1,)), ((), ())),
        preferred_element_type=jnp.float32,
        precision=lax.Precision.HIGHEST)
    out_ref[...] = s / denom


def _argmax_body(scores_ref, idx_ref, rows_ref):
    s = scores_ref[...]
    m = jnp.max(s, axis=0, keepdims=True)
    kiota = lax.broadcasted_iota(jnp.int32, (_K, _B), 0)
    cand = jnp.where(s == m, kiota, _K)
    idx = jnp.min(cand, axis=0, keepdims=True)
    idx_ref[...] = idx
    eye = (lax.broadcasted_iota(jnp.int32, (_B, _B), 0) ==
           lax.broadcasted_iota(jnp.int32, (_B, _B), 1)).astype(jnp.float32)
    idx_col = lax.dot_general(
        eye, idx.astype(jnp.float32), (((1,), (1,)), ((), ())),
        preferred_element_type=jnp.float32,
        precision=lax.Precision.HIGHEST)
    tiota = lax.broadcasted_iota(jnp.int32, (_B, _TOK), 1)
    rows_ref[...] = idx_col.astype(jnp.int32) * _TOK + tiota


@functools.lru_cache(maxsize=1)
def _make_sc_gather():
    NC, NS = 2, 16
    NW = NC * NS
    NROWS = _B * _TOK
    CH = 16
    rows_per_w = NROWS // NW
    n_ch = rows_per_w // CH

    @functools.partial(
        pl.kernel,
        out_type=jax.ShapeDtypeStruct((NROWS, _EMB), jnp.float32),
        mesh=plsc.VectorSubcoreMesh(core_axis_name="c", subcore_axis_name="s"),
        scratch_types=[
            pltpu.VMEM((rows_per_w,), jnp.int32),
            pltpu.VMEM((2, CH, _EMB), jnp.float32),
            pltpu.SemaphoreType.DMA,
            pltpu.SemaphoreType.DMA,
            pltpu.SemaphoreType.DMA,
            pltpu.SemaphoreType.DMA,
        ],
    )
    def gather_kernel(table_hbm, rows_hbm, out_hbm, idxall, buf, g0, g1, w0,
                      w1):
        gsem = (g0, g1)
        wsem = (w0, w1)
        wid = lax.axis_index("s") * NC + lax.axis_index("c")
        wbase = wid * rows_per_w
        pltpu.sync_copy(rows_hbm.at[pl.ds(wbase, rows_per_w)], idxall)

        def gather(j):
            return pltpu.async_copy(
                table_hbm.at[idxall.at[pl.ds(j * CH, CH)]],
                buf.at[j % 2], gsem[j % 2])

        def scatter(j):
            return pltpu.async_copy(
                buf.at[j % 2], out_hbm.at[pl.ds(wbase + j * CH, CH)],
                wsem[j % 2])

        gh = [None] * n_ch
        sh = [None] * n_ch
        gh[0] = gather(0)
        for j in range(n_ch):
            if j + 1 < n_ch:
                if j >= 1:
                    sh[j - 1].wait()
                gh[j + 1] = gather(j + 1)
            gh[j].wait()
            sh[j] = scatter(j)
        sh[n_ch - 2].wait()
        sh[n_ch - 1].wait()

    return gather_kernel


def kernel(emo_vec, image_tokens, W_emo, b_emo, W_img, b_img, style_dict):
    qc = pl.pallas_call(
        _query_body,
        grid=(_NTOK // _TB,),
        in_specs=[
            pl.BlockSpec((_B, _TB, _IMG_DIM), lambda g: (0, g, 0)),
            pl.BlockSpec((_B, _IN_DIM), lambda g: (0, 0)),
            pl.BlockSpec((_HALF, _IN_DIM), lambda g: (0, 0)),
            pl.BlockSpec((1, _HALF), lambda g: (0, 0)),
            pl.BlockSpec((_HALF, _IMG_DIM), lambda g: (0, 0)),
            pl.BlockSpec((1, _HALF), lambda g: (0, 0)),
        ],
        out_specs=pl.BlockSpec((_B, _EMB), lambda g: (0, 0)),
        out_shape=jax.ShapeDtypeStruct((_B, _EMB), jnp.float32),
        scratch_shapes=[pltpu.VMEM((_B, 8, _IMG_DIM), jnp.float32)],
    )(image_tokens, emo_vec, W_emo, b_emo.reshape(1, _HALF), W_img,
      b_img.reshape(1, _HALF))

    scores_t = pl.pallas_call(
        _scores_body,
        grid=(_K // _KB,),
        in_specs=[
            pl.BlockSpec((_KB, _TOK, _EMB), lambda g: (g, 0, 0)),
            pl.BlockSpec((_B, _EMB), lambda g: (0, 0)),
        ],
        out_specs=pl.BlockSpec((_KB, _B), lambda g: (g, 0)),
        out_shape=jax.ShapeDtypeStruct((_K, _B), jnp.float32),
    )(style_dict, qc)

    idx2d, rows2d = pl.pallas_call(
        _argmax_body,
        in_specs=[pl.BlockSpec((_K, _B), lambda: (0, 0))],
        out_specs=[pl.BlockSpec((1, _B), lambda: (0, 0)),
                   pl.BlockSpec((_B, _TOK), lambda: (0, 0))],
        out_shape=[jax.ShapeDtypeStruct((1, _B), jnp.int32),
                   jax.ShapeDtypeStruct((_B, _TOK), jnp.int32)],
    )(scores_t)

    indices = idx2d.reshape(_B)
    table = style_dict.reshape(_K * _TOK, _EMB)
    rows = rows2d.reshape(_B * _TOK)
    style = _make_sc_gather()(table, rows).reshape(_B, _TOK, _EMB)
    return style, indices


# speedup vs baseline: 1.9953x; 1.0444x over previous
import functools

import jax
import jax.numpy as jnp
from jax import lax
from jax.experimental import pallas as pl
from jax.experimental.pallas import tpu as pltpu
from jax.experimental.pallas import tpu_sc as plsc

_B = 64
_IN_DIM = 8
_IMG_DIM = 1152
_TOK = 64
_EMB = 2048
_K = 512
_NTOK = 576
_HALF = 1024

_TB = 48
_KB = 32


def _query_body(img_ref, emo_ref, wemo_ref, bemo_ref, wimg_ref, bimg_ref,
                q_ref, acc_ref):
    g = pl.program_id(0)

    @pl.when(g == 0)
    def _init():
        acc_ref[...] = jnp.zeros_like(acc_ref)

    s = img_ref[:, 0:8, :]
    for t in range(1, _TB // 8):
        s = s + img_ref[:, 8 * t:8 * (t + 1), :]
    acc_ref[...] += s

    @pl.when(g == pl.num_programs(0) - 1)
    def _epilogue():
        pooled = jnp.sum(acc_ref[...], axis=1) * (1.0 / _NTOK)
        q_img = lax.dot_general(
            pooled, wimg_ref[...], (((1,), (1,)), ((), ())),
            preferred_element_type=jnp.float32,
            precision=lax.Precision.HIGHEST) + bimg_ref[...]
        q_emo = lax.dot_general(
            emo_ref[...], wemo_ref[...], (((1,), (1,)), ((), ())),
            preferred_element_type=jnp.float32,
            precision=lax.Precision.HIGHEST) + bemo_ref[...]
        q = jnp.concatenate([q_emo, q_img], axis=1)
        q_ref[...] = q - jnp.mean(q, axis=1, keepdims=True)


def _scores_body(style_ref, q_ref, out_ref):
    x = style_ref[...]
    cs = jnp.sum(x, axis=1)
    norm = jnp.sqrt(jnp.sum(cs * cs, axis=1, keepdims=True))
    denom = jnp.maximum(norm, _TOK * 1e-6)
    s = lax.dot_general(
        cs, q_ref[...], (((1,), (1,)), ((), ())),
        preferred_element_type=jnp.float32,
        precision=lax.Precision.HIGHEST)
    out_ref[...] = s / denom


def _argmax_body(scores_ref, idx_ref, rows_ref):
    s = scores_ref[...]
    m = jnp.max(s, axis=0, keepdims=True)
    kiota = lax.broadcasted_iota(jnp.int32, (_K, _B), 0)
    cand = jnp.where(s == m, kiota, _K)
    idx = jnp.min(cand, axis=0, keepdims=True)
    idx_ref[...] = idx
    eye = (lax.broadcasted_iota(jnp.int32, (_B, _B), 0) ==
           lax.broadcasted_iota(jnp.int32, (_B, _B), 1)).astype(jnp.float32)
    idx_col = lax.dot_general(
        eye, idx.astype(jnp.float32), (((1,), (1,)), ((), ())),
        preferred_element_type=jnp.float32,
        precision=lax.Precision.HIGHEST)
    tiota = lax.broadcasted_iota(jnp.int32, (_B, _TOK), 1)
    rows_ref[...] = idx_col.astype(jnp.int32) * _TOK + tiota


@functools.lru_cache(maxsize=1)
def _make_sc_gather():
    NC, NS = 2, 16
    NW = NC * NS
    NROWS = _B * _TOK
    CH = 16
    rows_per_w = NROWS // NW
    n_ch = rows_per_w // CH

    @functools.partial(
        pl.kernel,
        out_type=jax.ShapeDtypeStruct((NROWS, _EMB), jnp.float32),
        mesh=plsc.VectorSubcoreMesh(core_axis_name="c", subcore_axis_name="s"),
        scratch_types=[
            pltpu.VMEM((rows_per_w,), jnp.int32),
            pltpu.VMEM((2, CH, _EMB), jnp.float32),
            pltpu.SemaphoreType.DMA,
            pltpu.SemaphoreType.DMA,
            pltpu.SemaphoreType.DMA,
            pltpu.SemaphoreType.DMA,
        ],
    )
    def gather_kernel(table_hbm, rows_hbm, out_hbm, idxall, buf, g0, g1, w0,
                      w1):
        gsem = (g0, g1)
        wsem = (w0, w1)
        wid = lax.axis_index("s") * NC + lax.axis_index("c")
        wbase = wid * rows_per_w
        pltpu.sync_copy(rows_hbm.at[pl.ds(wbase, rows_per_w)], idxall)

        def gather(j):
            return pltpu.async_copy(
                table_hbm.at[idxall.at[pl.ds(j * CH, CH)]],
                buf.at[j % 2], gsem[j % 2])

        def scatter(j):
            return pltpu.async_copy(
                buf.at[j % 2], out_hbm.at[pl.ds(wbase + j * CH, CH)],
                wsem[j % 2])

        gh = [None] * n_ch
        sh = [None] * n_ch
        gh[0] = gather(0)
        for j in range(n_ch):
            if j + 1 < n_ch:
                if j >= 1:
                    sh[j - 1].wait()
                gh[j + 1] = gather(j + 1)
            gh[j].wait()
            sh[j] = scatter(j)
        sh[n_ch - 2].wait()
        sh[n_ch - 1].wait()

    return gather_kernel


def kernel(emo_vec, image_tokens, W_emo, b_emo, W_img, b_img, style_dict):
    qc = pl.pallas_call(
        _query_body,
        grid=(_NTOK // _TB,),
        in_specs=[
            pl.BlockSpec((_B, _TB, _IMG_DIM), lambda g: (0, g, 0)),
            pl.BlockSpec((_B, _IN_DIM), lambda g: (0, 0)),
            pl.BlockSpec((_HALF, _IN_DIM), lambda g: (0, 0)),
            pl.BlockSpec((1, _HALF), lambda g: (0, 0)),
            pl.BlockSpec((_HALF, _IMG_DIM), lambda g: (0, 0)),
            pl.BlockSpec((1, _HALF), lambda g: (0, 0)),
        ],
        out_specs=pl.BlockSpec((_B, _EMB), lambda g: (0, 0)),
        out_shape=jax.ShapeDtypeStruct((_B, _EMB), jnp.float32),
        scratch_shapes=[pltpu.VMEM((_B, 8, _IMG_DIM), jnp.float32)],
    )(image_tokens, emo_vec, W_emo, b_emo.reshape(1, _HALF), W_img,
      b_img.reshape(1, _HALF))

    scores_t = pl.pallas_call(
        _scores_body,
        grid=(_K // _KB,),
        in_specs=[
            pl.BlockSpec((_KB, _TOK, _EMB), lambda g: (g, 0, 0)),
            pl.BlockSpec((_B, _EMB), lambda g: (0, 0)),
        ],
        out_specs=pl.BlockSpec((_KB, _B), lambda g: (g, 0)),
        out_shape=jax.ShapeDtypeStruct((_K, _B), jnp.float32),
    )(style_dict, qc)

    idx2d, rows2d = pl.pallas_call(
        _argmax_body,
        in_specs=[pl.BlockSpec((_K, _B), lambda: (0, 0))],
        out_specs=[pl.BlockSpec((1, _B), lambda: (0, 0)),
                   pl.BlockSpec((_B, _TOK), lambda: (0, 0))],
        out_shape=[jax.ShapeDtypeStruct((1, _B), jnp.int32),
                   jax.ShapeDtypeStruct((_B, _TOK), jnp.int32)],
    )(scores_t)

    indices = idx2d.reshape(_B)
    table = style_dict.reshape(_K * _TOK, _EMB)
    rows = rows2d.reshape(_B * _TOK)
    style = _make_sc_gather()(table, rows).reshape(_B, _TOK, _EMB)
    return style, indices
